# baseline (speedup 1.0000x reference)
import jax
import jax.numpy as jnp
import numpy as np
from jax.experimental import pallas as pl
from jax.experimental.pallas import tpu as pltpu

_N_FEAT = 256
_ALPHA = 0.95
_T = 8000
_C = 256
_NFULL = _T // _C
_CTAIL = _T - _NFULL * _C


def _ema_kernel(x_ref, a_ref, decay_ref, s0_ref, y_ref):
    a = a_ref[:]
    decay = decay_ref[:]
    s_row = s0_ref[:]
    for i in range(_NFULL + 1):
        lo = i * _C
        w = _C if i < _NFULL else _CTAIL
        x = x_ref[0, :, lo:lo + w]
        xt = jnp.transpose(x)
        ct = jnp.abs(xt) * (1.0 - _ALPHA)
        ut = jnp.dot(a[:w, :w], ct, preferred_element_type=jnp.float32)
        st = ut + decay[:w] * s_row
        y_ref[0, lo:lo + w, :] = xt * jax.lax.rsqrt(st)
        s_row = st[w - 1:w, :]


def _make(interpret=False):
    tj = np.arange(_C)
    powm = np.where(tj[:, None] >= tj[None, :],
                    _ALPHA ** (tj[:, None] - tj[None, :]), 0.0).astype(np.float32)
    decay = (_ALPHA ** (tj + 1.0)).astype(np.float32).reshape(_C, 1)

    def kfn(spec, unit_norm_state):
        B = spec.shape[0]
        s0 = jnp.reshape(unit_norm_state.astype(jnp.float32), (1, _N_FEAT))
        x3 = spec[:, 0, :_N_FEAT, :]
        out = pl.pallas_call(
            _ema_kernel,
            grid=(B,),
            in_specs=[
                pl.BlockSpec((1, _N_FEAT, _T), lambda b: (b, 0, 0)),
                pl.BlockSpec((_C, _C), lambda b: (0, 0)),
                pl.BlockSpec((_C, 1), lambda b: (0, 0)),
                pl.BlockSpec((1, _N_FEAT), lambda b: (0, 0)),
            ],
            out_specs=pl.BlockSpec((1, _T, _N_FEAT), lambda b: (b, 0, 0)),
            out_shape=jax.ShapeDtypeStruct((B, _T, _N_FEAT), jnp.float32),
            compiler_params=pltpu.CompilerParams(
                dimension_semantics=("parallel",),
                vmem_limit_bytes=48 * 1024 * 1024,
            ),
            name="ema_unit_norm",
            interpret=interpret,
        )(x3, jnp.asarray(powm), jnp.asarray(decay), s0)
        return jnp.swapaxes(out, 1, 2)[:, None]

    return kfn


def kernel(spec, unit_norm_state):
    return _make()(spec, unit_norm_state)


# speedup vs baseline: 1.6785x; 1.6785x over previous
import jax
import jax.numpy as jnp
import numpy as np
from jax.experimental import pallas as pl
from jax.experimental.pallas import tpu as pltpu

_N_FEAT = 256
_ALPHA = 0.95
_T = 8000
_C = 256
_NFULL = _T // _C
_CTAIL = _T - _NFULL * _C


def _ema_kernel(x_ref, a_ref, decay_ref, s0_ref, y_ref):
    a = a_ref[:]
    decay = decay_ref[:]
    s_row = s0_ref[:]
    for i in range(_NFULL + 1):
        lo = i * _C
        w = _C if i < _NFULL else _CTAIL
        x = x_ref[0, 0, :, lo:lo + w]
        xt = jnp.transpose(x)
        ct = jnp.abs(xt) * (1.0 - _ALPHA)
        ut = jnp.dot(a[:w, :w], ct, preferred_element_type=jnp.float32)
        st = ut + decay[:w] * s_row
        y_ref[0, 0, lo:lo + w, :] = xt * jax.lax.rsqrt(st)
        s_row = st[w - 1:w, :]


def _make(interpret=False):
    tj = np.arange(_C)
    powm = np.where(tj[:, None] >= tj[None, :],
                    _ALPHA ** (tj[:, None] - tj[None, :]), 0.0).astype(np.float32)
    decay = (_ALPHA ** (tj + 1.0)).astype(np.float32).reshape(_C, 1)

    def kfn(spec, unit_norm_state):
        B = spec.shape[0]
        s0 = jnp.reshape(unit_norm_state.astype(jnp.float32), (1, _N_FEAT))
        out = pl.pallas_call(
            _ema_kernel,
            grid=(B,),
            in_specs=[
                pl.BlockSpec((1, 1, _N_FEAT, _T), lambda b: (b, 0, 0, 0)),
                pl.BlockSpec((_C, _C), lambda b: (0, 0)),
                pl.BlockSpec((_C, 1), lambda b: (0, 0)),
                pl.BlockSpec((1, _N_FEAT), lambda b: (0, 0)),
            ],
            out_specs=pl.BlockSpec((1, 1, _T, _N_FEAT), lambda b: (b, 0, 0, 0)),
            out_shape=jax.ShapeDtypeStruct((B, 1, _T, _N_FEAT), jnp.float32),
            compiler_params=pltpu.CompilerParams(
                dimension_semantics=("parallel",),
                vmem_limit_bytes=48 * 1024 * 1024,
            ),
            name="ema_unit_norm",
            interpret=interpret,
        )(spec, jnp.asarray(powm), jnp.asarray(decay), s0)
        return jnp.swapaxes(out, 2, 3)

    return kfn


def kernel(spec, unit_norm_state):
    return _make()(spec, unit_norm_state)
